# speedup vs baseline: 1.5394x; 1.5394x over previous
import functools

import jax
import jax.numpy as jnp
from jax.experimental import pallas as pl

N = 10000
E = 320000
D = 128


def _pre_body(x_ref, W_ref, asrc_ref, adst_ref, h_ref, als_ref, ald_ref):
    h = x_ref[...] @ W_ref[...]
    h_ref[...] = h
    als_ref[...] = (h * asrc_ref[...]).sum(-1, keepdims=True)
    ald_ref[...] = (h * adst_ref[...]).sum(-1, keepdims=True)


def _pre(x, W, a_src, a_dst):
    return pl.pallas_call(
        _pre_body,
        out_shape=(
            jax.ShapeDtypeStruct((N, D), jnp.float32),
            jax.ShapeDtypeStruct((N, 1), jnp.float32),
            jax.ShapeDtypeStruct((N, 1), jnp.float32),
        ),
    )(x, W, a_src[None, :], a_dst[None, :])


def _edge_body(ea_ref, M_ref, out_ref):
    out_ref[...] = ea_ref[...] @ M_ref[...]


def _edge_al(edge_att, We1, a_e1, We2, a_e2):
    ve = jnp.stack([We1 @ a_e1, We2 @ a_e2], axis=-1)
    c = jax.lax.broadcasted_iota(jnp.int32, (D, 16), 0)
    j = jax.lax.broadcasted_iota(jnp.int32, (D, 16), 1)
    M = jnp.where(c // 16 == j // 2, jnp.tile(ve, (8, 8)), 0.0)
    ea = edge_att.reshape(E // 8, D)
    out = pl.pallas_call(
        _edge_body,
        out_shape=jax.ShapeDtypeStruct((E // 8, 16), jnp.float32),
    )(ea, M)
    out = out.reshape(E, 2)
    return out[:, 0], out[:, 1]


def _post_body(acc_ref, den_ref, b_ref, g_ref, beta_ref, o_ref, *, relu):
    den = den_ref[...]
    y = jnp.where(den > 0, acc_ref[...] / jnp.where(den > 0, den, 1.0), 0.0)
    y = y + b_ref[...]
    mu = y.mean(0, keepdims=True)
    var = ((y - mu) ** 2).mean(0, keepdims=True)
    y = (y - mu) * jax.lax.rsqrt(var + 1e-5) * g_ref[...] + beta_ref[...]
    if relu:
        y = jnp.maximum(y, 0.0)
    o_ref[...] = y


def _post(acc, den, b, g, beta, relu):
    return pl.pallas_call(
        functools.partial(_post_body, relu=relu),
        out_shape=jax.ShapeDtypeStruct((N, D), jnp.float32),
    )(acc, den[:, None], b[None, :], g[None, :], beta[None, :])


def _gat(h, als, ald, ale, src, dst):
    alpha = als[src] + ald[dst] + ale
    alpha = jnp.where(alpha > 0, alpha, 0.2 * alpha)
    ex = jnp.exp(alpha)
    den = jax.ops.segment_sum(ex, dst, num_segments=N)
    acc = jax.ops.segment_sum(h[src] * ex[:, None], dst, num_segments=N)
    return acc, den


def kernel(x, edge_index, edge_att, W1, a_src1, a_dst1, We1, a_e1, b1, W2,
           a_src2, a_dst2, We2, a_e2, b2, bn1_w, bn1_b, bn2_w, bn2_b):
    src, dst = edge_index[0], edge_index[1]
    ale1, ale2 = _edge_al(edge_att, We1, a_e1, We2, a_e2)
    h1, als1, ald1 = _pre(x, W1, a_src1, a_dst1)
    acc1, den1 = _gat(h1, als1[:, 0], ald1[:, 0], ale1, src, dst)
    y1 = _post(acc1, den1, b1, bn1_w, bn1_b, relu=True)
    h2, als2, ald2 = _pre(y1, W2, a_src2, a_dst2)
    acc2, den2 = _gat(h2, als2[:, 0], ald2[:, 0], ale2, src, dst)
    return _post(acc2, den2, b2, bn2_w, bn2_b, relu=False)


# speedup vs baseline: 8.7874x; 5.7084x over previous
import functools

import jax
import jax.numpy as jnp
from jax import lax
from jax.experimental import pallas as pl
from jax.experimental.pallas import tpu as pltpu
from jax.experimental.pallas import tpu_sc as plsc

N = 10000
E = 320000
D = 128

NC = 2
NS = 16
NW = NC * NS
RNG = 313
RPAD = 320
NPADDED = 10048

CHUNK = 512
CROWS = CHUNK // 128
KIN = 10000
NKIN = E // KIN
SUB = 25
NSUB = (KIN // 16) // SUB
CAPB = 1024
ECAP = 323584
SENT = -1.0e30

_mesh = plsc.VectorSubcoreMesh(
    core_axis_name="c", subcore_axis_name="s", num_cores=NC, num_subcores=NS)


def _wid():
    return lax.axis_index("s") * NC + lax.axis_index("c")


def _pre_body(x_ref, W_ref, asrc_ref, adst_ref, h_ref, als_ref, ald_ref):
    h = x_ref[...] @ W_ref[...]
    h_ref[...] = h
    als_ref[...] = (h * asrc_ref[...]).sum(-1, keepdims=True)
    ald_ref[...] = (h * adst_ref[...]).sum(-1, keepdims=True)


def _pre(x, W, a_src, a_dst):
    return pl.pallas_call(
        _pre_body,
        out_shape=(
            jax.ShapeDtypeStruct((N, D), jnp.float32),
            jax.ShapeDtypeStruct((N, 1), jnp.float32),
            jax.ShapeDtypeStruct((N, 1), jnp.float32),
        ),
    )(x, W, a_src[None, :], a_dst[None, :])


def _edge_body(ea_ref, M_ref, out_ref):
    out_ref[...] = ea_ref[...] @ M_ref[...]


def _edge_al(edge_att, We1, a_e1, We2, a_e2):
    ve = jnp.stack([We1 @ a_e1, We2 @ a_e2], axis=-1)
    c = lax.broadcasted_iota(jnp.int32, (D, 16), 0)
    j = lax.broadcasted_iota(jnp.int32, (D, 16), 1)
    M = jnp.where(c // 16 == j // 2, jnp.tile(ve, (8, 8)), 0.0)
    ea = edge_att.reshape(E // 8, D)
    out = pl.pallas_call(
        _edge_body,
        out_shape=jax.ShapeDtypeStruct((E // 8, 16), jnp.float32),
    )(ea, M)
    out = out.reshape(E, 2)
    return out[:, 0], out[:, 1]


def _post_body(acc_ref, den_ref, b_ref, g_ref, beta_ref, o_ref, *, relu):
    den = den_ref[...]
    y = jnp.where(den > 0, acc_ref[...] / jnp.where(den > 0, den, 1.0), 0.0)
    y = y + b_ref[...]
    mu = y.mean(0, keepdims=True)
    var = ((y - mu) ** 2).mean(0, keepdims=True)
    y = (y - mu) * lax.rsqrt(var + 1e-5) * g_ref[...] + beta_ref[...]
    if relu:
        y = jnp.maximum(y, 0.0)
    o_ref[...] = y


def _post(acc, den, b, g, beta, relu):
    return pl.pallas_call(
        functools.partial(_post_body, relu=relu),
        out_shape=jax.ShapeDtypeStruct((N, D), jnp.float32),
    )(acc, den[:, None], b[None, :], g[None, :], beta[None, :])


def _sentinel_fill(srcb, dstlb, ae1b, ae2b, lo, hi):
    z16 = jnp.zeros((16,), jnp.int32)
    s16 = jnp.full((16,), SENT, jnp.float32)

    def body(k, _):
        srcb[pl.ds(16 * k, 16)] = z16
        dstlb[pl.ds(16 * k, 16)] = z16
        ae1b[pl.ds(16 * k, 16)] = s16
        ae2b[pl.ds(16 * k, 16)] = s16
        return 0

    lax.fori_loop(lo // 16, hi // 16, body, 0)


def _filter_body(srcg, dstg, ae1, ae2, src_s, dstl_s, ae1_s, ae2_s, nch,
                 srcc0, srcc1, dstc0, dstc1, ae1c0, ae1c1, ae2c0, ae2c1,
                 srcb, dstlb, ae1b, ae2b, cvec, sem):
    wid = _wid()
    nbase = wid * RNG
    obase = wid * ECAP
    srccs, dstcs = (srcc0, srcc1), (dstc0, dstc1)
    ae1cs, ae2cs = (ae1c0, ae1c1), (ae2c0, ae2c1)

    _sentinel_fill(srcb, dstlb, ae1b, ae2b, 0, CAPB)

    def fire(c, b):
        o = c * KIN
        pltpu.make_async_copy(srcg.at[pl.ds(o, KIN)], srccs[b], sem).start()
        pltpu.make_async_copy(dstg.at[pl.ds(o, KIN)], dstcs[b], sem).start()
        pltpu.make_async_copy(ae1.at[pl.ds(o, KIN)], ae1cs[b], sem).start()
        pltpu.make_async_copy(ae2.at[pl.ds(o, KIN)], ae2cs[b], sem).start()

    def drain(c, b):
        o = c * KIN
        pltpu.make_async_copy(srcg.at[pl.ds(o, KIN)], srccs[b], sem).wait()
        pltpu.make_async_copy(dstg.at[pl.ds(o, KIN)], dstcs[b], sem).wait()
        pltpu.make_async_copy(ae1.at[pl.ds(o, KIN)], ae1cs[b], sem).wait()
        pltpu.make_async_copy(ae2.at[pl.ds(o, KIN)], ae2cs[b], sem).wait()

    fire(0, 0)
    fire(1, 1)

    def do_flush(nf):
        o = obase + nf * CHUNK
        pltpu.sync_copy(srcb, src_s.at[pl.ds(o, CAPB)])
        pltpu.sync_copy(dstlb, dstl_s.at[pl.ds(o, CAPB)])
        pltpu.sync_copy(ae1b, ae1_s.at[pl.ds(o, CAPB)])
        pltpu.sync_copy(ae2b, ae2_s.at[pl.ds(o, CAPB)])

        def shift_body(k, _):
            srcb[pl.ds(16 * k, 16)] = srcb[pl.ds(CHUNK + 16 * k, 16)]
            dstlb[pl.ds(16 * k, 16)] = dstlb[pl.ds(CHUNK + 16 * k, 16)]
            ae1b[pl.ds(16 * k, 16)] = ae1b[pl.ds(CHUNK + 16 * k, 16)]
            ae2b[pl.ds(16 * k, 16)] = ae2b[pl.ds(CHUNK + 16 * k, 16)]
            return 0

        lax.fori_loop(0, CHUNK // 16, shift_body, 0)
        _sentinel_fill(srcb, dstlb, ae1b, ae2b, CHUNK, CAPB)

    def flush(curv, nf):
        do = curv[0] >= CHUNK

        @pl.when(do)
        def _():
            do_flush(nf)

        curv = jnp.where(do, curv - CHUNK, curv)
        nf = jnp.where(do, nf + 1, nf)
        return curv, nf

    def pair_body(p, carry):
        curv, nf = carry
        for b in range(2):
            c = 2 * p + b
            drain(c, b)

            def sub_body(s, carry, b=b):
                cv, snf = carry

                def vec_body(k, cv, b=b):
                    i = s * SUB + k
                    srcv = srccs[b][pl.ds(16 * i, 16)]
                    dstv = dstcs[b][pl.ds(16 * i, 16)]
                    ae1v = ae1cs[b][pl.ds(16 * i, 16)]
                    ae2v = ae2cs[b][pl.ds(16 * i, 16)]
                    dstlv = dstv - nbase
                    m = (dstv >= nbase) & (dstlv < RNG)
                    pos = cv + jnp.cumsum(m.astype(jnp.int32)) - 1
                    pos = jnp.maximum(pos, 0)
                    plsc.store_scatter(srcb, [pos], srcv, mask=m)
                    plsc.store_scatter(dstlb, [pos], dstlv, mask=m)
                    plsc.store_scatter(ae1b, [pos], ae1v, mask=m)
                    plsc.store_scatter(ae2b, [pos], ae2v, mask=m)
                    return cv + plsc.all_reduce_population_count(m)

                cv = lax.fori_loop(0, SUB, vec_body, cv)
                return flush(cv, snf)

            curv, nf = lax.fori_loop(0, NSUB, sub_body, (curv, nf))

            @pl.when(c + 2 < NKIN)
            def _():
                fire(c + 2, b)
        return curv, nf

    curv = jnp.zeros((16,), jnp.int32)
    nf = jnp.int32(0)
    curv, nf = lax.fori_loop(0, NKIN // 2, pair_body, (curv, nf))

    for _ in range(2):
        do = curv[0] > 0

        @pl.when(do)
        def _():
            do_flush(nf)

        curv = jnp.where(do, jnp.maximum(curv - CHUNK, 0), curv)
        nf = jnp.where(do, nf + 1, nf)

    cvec[...] = jnp.zeros((16,), jnp.int32) + nf
    pltpu.sync_copy(cvec, nch.at[pl.ds(wid * 16, 16)])


_filter_call = pl.kernel(
    _filter_body,
    out_type=(
        jax.ShapeDtypeStruct((NW * ECAP,), jnp.int32),
        jax.ShapeDtypeStruct((NW * ECAP,), jnp.int32),
        jax.ShapeDtypeStruct((NW * ECAP,), jnp.float32),
        jax.ShapeDtypeStruct((NW * ECAP,), jnp.float32),
        jax.ShapeDtypeStruct((NW * 16,), jnp.int32),
    ),
    mesh=_mesh,
    compiler_params=pltpu.CompilerParams(needs_layout_passes=False),
    scratch_types=[
        pltpu.VMEM((KIN,), jnp.int32),
        pltpu.VMEM((KIN,), jnp.int32),
        pltpu.VMEM((KIN,), jnp.int32),
        pltpu.VMEM((KIN,), jnp.int32),
        pltpu.VMEM((KIN,), jnp.float32),
        pltpu.VMEM((KIN,), jnp.float32),
        pltpu.VMEM((KIN,), jnp.float32),
        pltpu.VMEM((KIN,), jnp.float32),
        pltpu.VMEM((CAPB,), jnp.int32),
        pltpu.VMEM((CAPB,), jnp.int32),
        pltpu.VMEM((CAPB,), jnp.float32),
        pltpu.VMEM((CAPB,), jnp.float32),
        pltpu.VMEM((16,), jnp.int32),
        pltpu.SemaphoreType.DMA,
    ],
)


def _layer_body(h, als, aldp, src_s, dstl_s, ae_s, nch, acc_o, den_o,
                als_t, ald_l, acc, den_l, srcfl, srcb, dstlb, aeb, exb,
                rows0, rows1, rows2, rows3, cvec, sem):
    rows_bufs = (rows0, rows1, rows2, rows3)
    wid = _wid()
    nbase = wid * RNG
    abase = (nbase // 8) * 8
    adj = nbase - abase
    obase = wid * ECAP

    pltpu.sync_copy(als, als_t)
    pltpu.sync_copy(aldp.at[pl.ds(abase, RPAD + 8)], ald_l)
    pltpu.sync_copy(nch.at[pl.ds(wid * 16, 16)], cvec)
    nchunks = cvec[...][0]

    z16 = jnp.zeros((16,), jnp.float32)

    def zero_body(r, _):
        for v in range(8):
            acc[pl.ds(r * D + 16 * v, 16)] = z16
        den_l[pl.ds(r * 16, 16)] = z16
        return 0

    lax.fori_loop(0, RPAD, zero_body, 0)

    def chunk_body(c, _):
        o = obase + c * CHUNK
        pltpu.sync_copy(src_s.at[pl.ds(o, CHUNK)], srcfl)
        pltpu.sync_copy(dstl_s.at[pl.ds(o, CHUNK)], dstlb)
        pltpu.sync_copy(ae_s.at[pl.ds(o, CHUNK)], aeb)

        for k in range(CHUNK // 16):
            srcb[k // 8, pl.ds(16 * (k % 8), 16)] = srcfl[pl.ds(16 * k, 16)]

        for j in range(CROWS):
            pltpu.make_async_copy(
                h.at[srcb.at[j]], rows_bufs[j], sem).start()

        def ex_body(i, _):
            srcv = srcfl[pl.ds(16 * i, 16)]
            dstlv = dstlb[pl.ds(16 * i, 16)]
            aev = aeb[pl.ds(16 * i, 16)]
            alsv = plsc.load_gather(als_t, [srcv])
            aldv = plsc.load_gather(ald_l, [dstlv + adj])
            a = alsv + aldv + aev
            a = jnp.where(a > 0, a, 0.2 * a)
            exb[pl.ds(16 * i, 16)] = jnp.exp(a)
            return 0

        lax.fori_loop(0, CHUNK // 16, ex_body, 0)

        for j in range(CROWS):
            pltpu.make_async_copy(
                h.at[srcb.at[j]], rows_bufs[j], sem).wait()

        for j in range(CROWS):
            rj = rows_bufs[j]

            def edge_block(i, _, j=j, rj=rj):
                dv = dstlb[pl.ds(128 * j + 16 * i, 16)]
                sv = exb[pl.ds(128 * j + 16 * i, 16)]
                for lane in range(16):
                    e = 16 * i + lane
                    d = dv[lane]
                    s = sv[lane]
                    for v in range(8):
                        plsc.addupdate(acc.at[pl.ds(d * D + 16 * v, 16)],
                                       rj[e, pl.ds(16 * v, 16)] * s)
                    plsc.addupdate(den_l.at[pl.ds(d * 16, 16)],
                                   jnp.zeros((16,), jnp.float32) + s)
                return 0

            lax.fori_loop(0, 128 // 16, edge_block, 0)
        return 0

    lax.fori_loop(0, nchunks, chunk_body, 0)

    pltpu.sync_copy(acc, acc_o.at[pl.ds(wid * RPAD * D, RPAD * D)])
    pltpu.sync_copy(den_l, den_o.at[pl.ds(wid * RPAD * 16, RPAD * 16)])


_layer_call = pl.kernel(
    _layer_body,
    out_type=(
        jax.ShapeDtypeStruct((NW * RPAD * D,), jnp.float32),
        jax.ShapeDtypeStruct((NW * RPAD * 16,), jnp.float32),
    ),
    mesh=_mesh,
    compiler_params=pltpu.CompilerParams(needs_layout_passes=False),
    scratch_types=[
        pltpu.VMEM((N,), jnp.float32),
        pltpu.VMEM((RPAD + 8,), jnp.float32),
        pltpu.VMEM((RPAD * D,), jnp.float32),
        pltpu.VMEM((RPAD * 16,), jnp.float32),
        pltpu.VMEM((CHUNK,), jnp.int32),
        pltpu.VMEM((CROWS, 128), jnp.int32),
        pltpu.VMEM((CHUNK,), jnp.int32),
        pltpu.VMEM((CHUNK,), jnp.float32),
        pltpu.VMEM((CHUNK,), jnp.float32),
        pltpu.VMEM((128, D), jnp.float32),
        pltpu.VMEM((128, D), jnp.float32),
        pltpu.VMEM((128, D), jnp.float32),
        pltpu.VMEM((128, D), jnp.float32),
        pltpu.VMEM((16,), jnp.int32),
        pltpu.SemaphoreType.DMA,
    ],
)


def _assemble(acc, den):
    accr = acc.reshape(NW, RPAD, D)[:, :RNG].reshape(NW * RNG, D)[:N]
    denr = den.reshape(NW, RPAD, 16)[:, :RNG, 0].reshape(NW * RNG)[:N]
    return accr, denr


def kernel(x, edge_index, edge_att, W1, a_src1, a_dst1, We1, a_e1, b1, W2,
           a_src2, a_dst2, We2, a_e2, b2, bn1_w, bn1_b, bn2_w, bn2_b):
    ale1, ale2 = _edge_al(edge_att, We1, a_e1, We2, a_e2)
    src_s, dstl_s, ae1_s, ae2_s, nch = _filter_call(
        edge_index[0], edge_index[1], ale1, ale2)

    h1, als1, ald1 = _pre(x, W1, a_src1, a_dst1)
    ald1p = jnp.pad(ald1[:, 0], (0, NPADDED - N))
    acc1, den1 = _layer_call(h1, als1[:, 0], ald1p, src_s, dstl_s, ae1_s, nch)
    acc1r, den1r = _assemble(acc1, den1)
    y1 = _post(acc1r, den1r, b1, bn1_w, bn1_b, relu=True)

    h2, als2, ald2 = _pre(y1, W2, a_src2, a_dst2)
    ald2p = jnp.pad(ald2[:, 0], (0, NPADDED - N))
    acc2, den2 = _layer_call(h2, als2[:, 0], ald2p, src_s, dstl_s, ae2_s, nch)
    acc2r, den2r = _assemble(acc2, den2)
    return _post(acc2r, den2r, b2, bn2_w, bn2_b, relu=False)


# speedup vs baseline: 8.8099x; 1.0026x over previous
import functools

import jax
import jax.numpy as jnp
from jax import lax
from jax.experimental import pallas as pl
from jax.experimental.pallas import tpu as pltpu
from jax.experimental.pallas import tpu_sc as plsc

N = 10000
E = 320000
D = 128

NC = 2
NS = 16
NW = NC * NS
RNG = 313
RPAD = 320
NPADDED = 10048

CHUNK = 512
CROWS = CHUNK // 128
KIN = 10000
NKIN = E // KIN
SUB = 25
NSUB = (KIN // 16) // SUB
CAPB = 1024
ECAP = 323584
SENT = -1.0e30

_mesh = plsc.VectorSubcoreMesh(
    core_axis_name="c", subcore_axis_name="s", num_cores=NC, num_subcores=NS)


def _wid():
    return lax.axis_index("s") * NC + lax.axis_index("c")


def _pre_body(x_ref, W_ref, asrc_ref, adst_ref, h_ref, als_ref, ald_ref):
    h = x_ref[...] @ W_ref[...]
    h_ref[...] = h
    als_ref[...] = (h * asrc_ref[...]).sum(-1, keepdims=True)
    ald_ref[...] = (h * adst_ref[...]).sum(-1, keepdims=True)


def _pre(x, W, a_src, a_dst):
    return pl.pallas_call(
        _pre_body,
        out_shape=(
            jax.ShapeDtypeStruct((N, D), jnp.float32),
            jax.ShapeDtypeStruct((N, 1), jnp.float32),
            jax.ShapeDtypeStruct((N, 1), jnp.float32),
        ),
    )(x, W, a_src[None, :], a_dst[None, :])


def _edge_body(ea_ref, M_ref, out_ref):
    out_ref[...] = ea_ref[...] @ M_ref[...]


def _edge_al(edge_att, We1, a_e1, We2, a_e2):
    ve = jnp.stack([We1 @ a_e1, We2 @ a_e2], axis=-1)
    c = lax.broadcasted_iota(jnp.int32, (D, 16), 0)
    j = lax.broadcasted_iota(jnp.int32, (D, 16), 1)
    M = jnp.where(c // 16 == j // 2, jnp.tile(ve, (8, 8)), 0.0)
    ea = edge_att.reshape(E // 8, D)
    out = pl.pallas_call(
        _edge_body,
        out_shape=jax.ShapeDtypeStruct((E // 8, 16), jnp.float32),
    )(ea, M)
    out = out.reshape(E, 2)
    return out[:, 0], out[:, 1]


def _post_body(acc_ref, den_ref, b_ref, g_ref, beta_ref, o_ref, *, relu):
    den = den_ref[...]
    y = jnp.where(den > 0, acc_ref[...] / jnp.where(den > 0, den, 1.0), 0.0)
    y = y + b_ref[...]
    mu = y.mean(0, keepdims=True)
    var = ((y - mu) ** 2).mean(0, keepdims=True)
    y = (y - mu) * lax.rsqrt(var + 1e-5) * g_ref[...] + beta_ref[...]
    if relu:
        y = jnp.maximum(y, 0.0)
    o_ref[...] = y


def _post(acc, den, b, g, beta, relu):
    return pl.pallas_call(
        functools.partial(_post_body, relu=relu),
        out_shape=jax.ShapeDtypeStruct((N, D), jnp.float32),
    )(acc, den[:, None], b[None, :], g[None, :], beta[None, :])


def _sentinel_fill(srcb, dstlb, ae1b, ae2b, lo, hi):
    z16 = jnp.zeros((16,), jnp.int32)
    s16 = jnp.full((16,), SENT, jnp.float32)

    def body(k, _):
        srcb[pl.ds(16 * k, 16)] = z16
        dstlb[pl.ds(16 * k, 16)] = z16
        ae1b[pl.ds(16 * k, 16)] = s16
        ae2b[pl.ds(16 * k, 16)] = s16
        return 0

    lax.fori_loop(lo // 16, hi // 16, body, 0)


def _filter_body(srcg, dstg, ae1, ae2, src_s, dstl_s, ae1_s, ae2_s, nch,
                 srcc0, srcc1, dstc0, dstc1, ae1c0, ae1c1, ae2c0, ae2c1,
                 srcb, dstlb, ae1b, ae2b, cvec, sem):
    wid = _wid()
    nbase = wid * RNG
    obase = wid * ECAP
    srccs, dstcs = (srcc0, srcc1), (dstc0, dstc1)
    ae1cs, ae2cs = (ae1c0, ae1c1), (ae2c0, ae2c1)

    _sentinel_fill(srcb, dstlb, ae1b, ae2b, 0, CAPB)

    def fire(c, b):
        o = c * KIN
        pltpu.make_async_copy(srcg.at[pl.ds(o, KIN)], srccs[b], sem).start()
        pltpu.make_async_copy(dstg.at[pl.ds(o, KIN)], dstcs[b], sem).start()
        pltpu.make_async_copy(ae1.at[pl.ds(o, KIN)], ae1cs[b], sem).start()
        pltpu.make_async_copy(ae2.at[pl.ds(o, KIN)], ae2cs[b], sem).start()

    def drain(c, b):
        o = c * KIN
        pltpu.make_async_copy(srcg.at[pl.ds(o, KIN)], srccs[b], sem).wait()
        pltpu.make_async_copy(dstg.at[pl.ds(o, KIN)], dstcs[b], sem).wait()
        pltpu.make_async_copy(ae1.at[pl.ds(o, KIN)], ae1cs[b], sem).wait()
        pltpu.make_async_copy(ae2.at[pl.ds(o, KIN)], ae2cs[b], sem).wait()

    fire(0, 0)
    fire(1, 1)

    def do_flush(nf):
        o = obase + nf * CHUNK
        pltpu.sync_copy(srcb, src_s.at[pl.ds(o, CAPB)])
        pltpu.sync_copy(dstlb, dstl_s.at[pl.ds(o, CAPB)])
        pltpu.sync_copy(ae1b, ae1_s.at[pl.ds(o, CAPB)])
        pltpu.sync_copy(ae2b, ae2_s.at[pl.ds(o, CAPB)])

        def shift_body(k, _):
            srcb[pl.ds(16 * k, 16)] = srcb[pl.ds(CHUNK + 16 * k, 16)]
            dstlb[pl.ds(16 * k, 16)] = dstlb[pl.ds(CHUNK + 16 * k, 16)]
            ae1b[pl.ds(16 * k, 16)] = ae1b[pl.ds(CHUNK + 16 * k, 16)]
            ae2b[pl.ds(16 * k, 16)] = ae2b[pl.ds(CHUNK + 16 * k, 16)]
            return 0

        lax.fori_loop(0, CHUNK // 16, shift_body, 0)
        _sentinel_fill(srcb, dstlb, ae1b, ae2b, CHUNK, CAPB)

    def flush(curv, nf):
        do = curv[0] >= CHUNK

        @pl.when(do)
        def _():
            do_flush(nf)

        curv = jnp.where(do, curv - CHUNK, curv)
        nf = jnp.where(do, nf + 1, nf)
        return curv, nf

    def pair_body(p, carry):
        curv, nf = carry
        for b in range(2):
            c = 2 * p + b
            drain(c, b)

            def sub_body(s, carry, b=b):
                cv, snf = carry

                def vec_body(k, cv, b=b):
                    i = s * SUB + k
                    srcv = srccs[b][pl.ds(16 * i, 16)]
                    dstv = dstcs[b][pl.ds(16 * i, 16)]
                    ae1v = ae1cs[b][pl.ds(16 * i, 16)]
                    ae2v = ae2cs[b][pl.ds(16 * i, 16)]
                    dstlv = dstv - nbase
                    m = (dstv >= nbase) & (dstlv < RNG)
                    pos = cv + jnp.cumsum(m.astype(jnp.int32)) - 1
                    pos = jnp.maximum(pos, 0)
                    plsc.store_scatter(srcb, [pos], srcv, mask=m)
                    plsc.store_scatter(dstlb, [pos], dstlv, mask=m)
                    plsc.store_scatter(ae1b, [pos], ae1v, mask=m)
                    plsc.store_scatter(ae2b, [pos], ae2v, mask=m)
                    return cv + plsc.all_reduce_population_count(m)

                cv = lax.fori_loop(0, SUB, vec_body, cv)
                return flush(cv, snf)

            curv, nf = lax.fori_loop(0, NSUB, sub_body, (curv, nf))

            @pl.when(c + 2 < NKIN)
            def _():
                fire(c + 2, b)
        return curv, nf

    curv = jnp.zeros((16,), jnp.int32)
    nf = jnp.int32(0)
    curv, nf = lax.fori_loop(0, NKIN // 2, pair_body, (curv, nf))

    for _ in range(2):
        do = curv[0] > 0

        @pl.when(do)
        def _():
            do_flush(nf)

        curv = jnp.where(do, jnp.maximum(curv - CHUNK, 0), curv)
        nf = jnp.where(do, nf + 1, nf)

    cvec[...] = jnp.zeros((16,), jnp.int32) + nf
    pltpu.sync_copy(cvec, nch.at[pl.ds(wid * 16, 16)])


_filter_call = pl.kernel(
    _filter_body,
    out_type=(
        jax.ShapeDtypeStruct((NW * ECAP,), jnp.int32),
        jax.ShapeDtypeStruct((NW * ECAP,), jnp.int32),
        jax.ShapeDtypeStruct((NW * ECAP,), jnp.float32),
        jax.ShapeDtypeStruct((NW * ECAP,), jnp.float32),
        jax.ShapeDtypeStruct((NW * 16,), jnp.int32),
    ),
    mesh=_mesh,
    compiler_params=pltpu.CompilerParams(needs_layout_passes=False),
    scratch_types=[
        pltpu.VMEM((KIN,), jnp.int32),
        pltpu.VMEM((KIN,), jnp.int32),
        pltpu.VMEM((KIN,), jnp.int32),
        pltpu.VMEM((KIN,), jnp.int32),
        pltpu.VMEM((KIN,), jnp.float32),
        pltpu.VMEM((KIN,), jnp.float32),
        pltpu.VMEM((KIN,), jnp.float32),
        pltpu.VMEM((KIN,), jnp.float32),
        pltpu.VMEM((CAPB,), jnp.int32),
        pltpu.VMEM((CAPB,), jnp.int32),
        pltpu.VMEM((CAPB,), jnp.float32),
        pltpu.VMEM((CAPB,), jnp.float32),
        pltpu.VMEM((16,), jnp.int32),
        pltpu.SemaphoreType.DMA,
    ],
)


def _layer_body(h, als, aldp, src_s, dstl_s, ae_s, nch, acc_o, den_o,
                als_t, ald_l, acc, den_l, srcfl, srcb, dstlb, aeb, exb,
                rows0, rows1, rows2, rows3, cvec, sem0, sem1, sem2, sem3):
    rows_bufs = (rows0, rows1, rows2, rows3)
    sems = (sem0, sem1, sem2, sem3)
    wid = _wid()
    nbase = wid * RNG
    abase = (nbase // 8) * 8
    adj = nbase - abase
    obase = wid * ECAP

    pltpu.sync_copy(als, als_t)
    pltpu.sync_copy(aldp.at[pl.ds(abase, RPAD + 8)], ald_l)
    pltpu.sync_copy(nch.at[pl.ds(wid * 16, 16)], cvec)
    nchunks = cvec[...][0]

    z16 = jnp.zeros((16,), jnp.float32)

    def zero_body(r, _):
        for v in range(8):
            acc[pl.ds(r * D + 16 * v, 16)] = z16
        den_l[pl.ds(r * 16, 16)] = z16
        return 0

    lax.fori_loop(0, RPAD, zero_body, 0)

    def chunk_body(c, _):
        o = obase + c * CHUNK
        pltpu.sync_copy(src_s.at[pl.ds(o, CHUNK)], srcfl)
        pltpu.sync_copy(dstl_s.at[pl.ds(o, CHUNK)], dstlb)
        pltpu.sync_copy(ae_s.at[pl.ds(o, CHUNK)], aeb)

        for k in range(CHUNK // 16):
            srcb[k // 8, pl.ds(16 * (k % 8), 16)] = srcfl[pl.ds(16 * k, 16)]

        for j in range(CROWS):
            pltpu.make_async_copy(
                h.at[srcb.at[j]], rows_bufs[j], sems[j]).start()

        def ex_body(i, _):
            srcv = srcfl[pl.ds(16 * i, 16)]
            dstlv = dstlb[pl.ds(16 * i, 16)]
            aev = aeb[pl.ds(16 * i, 16)]
            alsv = plsc.load_gather(als_t, [srcv])
            aldv = plsc.load_gather(ald_l, [dstlv + adj])
            a = alsv + aldv + aev
            a = jnp.where(a > 0, a, 0.2 * a)
            exb[pl.ds(16 * i, 16)] = jnp.exp(a)
            return 0

        lax.fori_loop(0, CHUNK // 16, ex_body, 0)

        for j in range(CROWS):
            pltpu.make_async_copy(
                h.at[srcb.at[j]], rows_bufs[j], sems[j]).wait()
            rj = rows_bufs[j]

            def edge_block(i, _, j=j, rj=rj):
                dv = dstlb[pl.ds(128 * j + 16 * i, 16)]
                sv = exb[pl.ds(128 * j + 16 * i, 16)]
                for lane in range(16):
                    e = 16 * i + lane
                    d = dv[lane]
                    s = sv[lane]
                    for v in range(8):
                        plsc.addupdate(acc.at[pl.ds(d * D + 16 * v, 16)],
                                       rj[e, pl.ds(16 * v, 16)] * s)
                    plsc.addupdate(den_l.at[pl.ds(d * 16, 16)],
                                   jnp.zeros((16,), jnp.float32) + s)
                return 0

            lax.fori_loop(0, 128 // 16, edge_block, 0)
        return 0

    lax.fori_loop(0, nchunks, chunk_body, 0)

    pltpu.sync_copy(acc, acc_o.at[pl.ds(wid * RPAD * D, RPAD * D)])
    pltpu.sync_copy(den_l, den_o.at[pl.ds(wid * RPAD * 16, RPAD * 16)])


_layer_call = pl.kernel(
    _layer_body,
    out_type=(
        jax.ShapeDtypeStruct((NW * RPAD * D,), jnp.float32),
        jax.ShapeDtypeStruct((NW * RPAD * 16,), jnp.float32),
    ),
    mesh=_mesh,
    compiler_params=pltpu.CompilerParams(needs_layout_passes=False),
    scratch_types=[
        pltpu.VMEM((N,), jnp.float32),
        pltpu.VMEM((RPAD + 8,), jnp.float32),
        pltpu.VMEM((RPAD * D,), jnp.float32),
        pltpu.VMEM((RPAD * 16,), jnp.float32),
        pltpu.VMEM((CHUNK,), jnp.int32),
        pltpu.VMEM((CROWS, 128), jnp.int32),
        pltpu.VMEM((CHUNK,), jnp.int32),
        pltpu.VMEM((CHUNK,), jnp.float32),
        pltpu.VMEM((CHUNK,), jnp.float32),
        pltpu.VMEM((128, D), jnp.float32),
        pltpu.VMEM((128, D), jnp.float32),
        pltpu.VMEM((128, D), jnp.float32),
        pltpu.VMEM((128, D), jnp.float32),
        pltpu.VMEM((16,), jnp.int32),
        pltpu.SemaphoreType.DMA,
        pltpu.SemaphoreType.DMA,
        pltpu.SemaphoreType.DMA,
        pltpu.SemaphoreType.DMA,
    ],
)


def _assemble(acc, den):
    accr = acc.reshape(NW, RPAD, D)[:, :RNG].reshape(NW * RNG, D)[:N]
    denr = den.reshape(NW, RPAD, 16)[:, :RNG, 0].reshape(NW * RNG)[:N]
    return accr, denr


def kernel(x, edge_index, edge_att, W1, a_src1, a_dst1, We1, a_e1, b1, W2,
           a_src2, a_dst2, We2, a_e2, b2, bn1_w, bn1_b, bn2_w, bn2_b):
    ale1, ale2 = _edge_al(edge_att, We1, a_e1, We2, a_e2)
    src_s, dstl_s, ae1_s, ae2_s, nch = _filter_call(
        edge_index[0], edge_index[1], ale1, ale2)

    h1, als1, ald1 = _pre(x, W1, a_src1, a_dst1)
    ald1p = jnp.pad(ald1[:, 0], (0, NPADDED - N))
    acc1, den1 = _layer_call(h1, als1[:, 0], ald1p, src_s, dstl_s, ae1_s, nch)
    acc1r, den1r = _assemble(acc1, den1)
    y1 = _post(acc1r, den1r, b1, bn1_w, bn1_b, relu=True)

    h2, als2, ald2 = _pre(y1, W2, a_src2, a_dst2)
    ald2p = jnp.pad(ald2[:, 0], (0, NPADDED - N))
    acc2, den2 = _layer_call(h2, als2[:, 0], ald2p, src_s, dstl_s, ae2_s, nch)
    acc2r, den2r = _assemble(acc2, den2)
    return _post(acc2r, den2r, b2, bn2_w, bn2_b, relu=False)


# speedup vs baseline: 8.9383x; 1.0146x over previous
import functools

import jax
import jax.numpy as jnp
from jax import lax
from jax.experimental import pallas as pl
from jax.experimental.pallas import tpu as pltpu
from jax.experimental.pallas import tpu_sc as plsc

N = 10000
E = 320000
D = 128

NC = 2
NS = 16
NW = NC * NS
RNG = 313
RPAD = 320
NPADDED = 10048

CHUNK = 512
CROWS = CHUNK // 128
KIN = 10000
NKIN = E // KIN
SUB = 25
NSUB = (KIN // 16) // SUB
CAPB = 1024
ECAP = 323584
SENT = -1.0e30

_mesh = plsc.VectorSubcoreMesh(
    core_axis_name="c", subcore_axis_name="s", num_cores=NC, num_subcores=NS)


def _wid():
    return lax.axis_index("s") * NC + lax.axis_index("c")


def _pre_body(x_ref, W_ref, asrc_ref, adst_ref, h_ref, als_ref, ald_ref):
    h = x_ref[...] @ W_ref[...]
    h_ref[...] = h
    als_ref[...] = (h * asrc_ref[...]).sum(-1, keepdims=True)
    ald_ref[...] = (h * adst_ref[...]).sum(-1, keepdims=True)


def _pre(x, W, a_src, a_dst):
    return pl.pallas_call(
        _pre_body,
        out_shape=(
            jax.ShapeDtypeStruct((N, D), jnp.float32),
            jax.ShapeDtypeStruct((N, 1), jnp.float32),
            jax.ShapeDtypeStruct((N, 1), jnp.float32),
        ),
    )(x, W, a_src[None, :], a_dst[None, :])


def _edge_body(ea_ref, M_ref, out_ref):
    out_ref[...] = ea_ref[...] @ M_ref[...]


def _edge_al(edge_att, We1, a_e1, We2, a_e2):
    ve = jnp.stack([We1 @ a_e1, We2 @ a_e2], axis=-1)
    c = lax.broadcasted_iota(jnp.int32, (D, 16), 0)
    j = lax.broadcasted_iota(jnp.int32, (D, 16), 1)
    M = jnp.where(c // 16 == j // 2, jnp.tile(ve, (8, 8)), 0.0)
    ea = edge_att.reshape(E // 8, D)
    out = pl.pallas_call(
        _edge_body,
        out_shape=jax.ShapeDtypeStruct((E // 8, 16), jnp.float32),
    )(ea, M)
    out = out.reshape(E, 2)
    return out[:, 0], out[:, 1]


def _post_body(acc_ref, den_ref, b_ref, g_ref, beta_ref, o_ref, *, relu):
    den = den_ref[...]
    y = jnp.where(den > 0, acc_ref[...] / jnp.where(den > 0, den, 1.0), 0.0)
    y = y + b_ref[...]
    mu = y.mean(0, keepdims=True)
    var = ((y - mu) ** 2).mean(0, keepdims=True)
    y = (y - mu) * lax.rsqrt(var + 1e-5) * g_ref[...] + beta_ref[...]
    if relu:
        y = jnp.maximum(y, 0.0)
    o_ref[...] = y


def _post(acc, den, b, g, beta, relu):
    return pl.pallas_call(
        functools.partial(_post_body, relu=relu),
        out_shape=jax.ShapeDtypeStruct((N, D), jnp.float32),
    )(acc, den[:, None], b[None, :], g[None, :], beta[None, :])


def _sentinel_fill(srcb, dstlb, ae1b, ae2b, lo, hi):
    z16 = jnp.zeros((16,), jnp.int32)
    s16 = jnp.full((16,), SENT, jnp.float32)

    def body(k, _):
        srcb[pl.ds(16 * k, 16)] = z16
        dstlb[pl.ds(16 * k, 16)] = z16
        ae1b[pl.ds(16 * k, 16)] = s16
        ae2b[pl.ds(16 * k, 16)] = s16
        return 0

    lax.fori_loop(lo // 16, hi // 16, body, 0)


def _filter_body(srcg, dstg, ae1, ae2, src_s, dstl_s, ae1_s, ae2_s, nch,
                 srcc0, srcc1, dstc0, dstc1, ae1c0, ae1c1, ae2c0, ae2c1,
                 srcb, dstlb, ae1b, ae2b, cvec, sem):
    wid = _wid()
    nbase = wid * RNG
    obase = wid * ECAP
    srccs, dstcs = (srcc0, srcc1), (dstc0, dstc1)
    ae1cs, ae2cs = (ae1c0, ae1c1), (ae2c0, ae2c1)

    _sentinel_fill(srcb, dstlb, ae1b, ae2b, 0, CAPB)

    def fire(c, b):
        o = c * KIN
        pltpu.make_async_copy(srcg.at[pl.ds(o, KIN)], srccs[b], sem).start()
        pltpu.make_async_copy(dstg.at[pl.ds(o, KIN)], dstcs[b], sem).start()
        pltpu.make_async_copy(ae1.at[pl.ds(o, KIN)], ae1cs[b], sem).start()
        pltpu.make_async_copy(ae2.at[pl.ds(o, KIN)], ae2cs[b], sem).start()

    def drain(c, b):
        o = c * KIN
        pltpu.make_async_copy(srcg.at[pl.ds(o, KIN)], srccs[b], sem).wait()
        pltpu.make_async_copy(dstg.at[pl.ds(o, KIN)], dstcs[b], sem).wait()
        pltpu.make_async_copy(ae1.at[pl.ds(o, KIN)], ae1cs[b], sem).wait()
        pltpu.make_async_copy(ae2.at[pl.ds(o, KIN)], ae2cs[b], sem).wait()

    fire(0, 0)
    fire(1, 1)

    def do_flush(nf):
        o = obase + nf * CHUNK
        pltpu.sync_copy(srcb, src_s.at[pl.ds(o, CAPB)])
        pltpu.sync_copy(dstlb, dstl_s.at[pl.ds(o, CAPB)])
        pltpu.sync_copy(ae1b, ae1_s.at[pl.ds(o, CAPB)])
        pltpu.sync_copy(ae2b, ae2_s.at[pl.ds(o, CAPB)])

        def shift_body(k, _):
            srcb[pl.ds(16 * k, 16)] = srcb[pl.ds(CHUNK + 16 * k, 16)]
            dstlb[pl.ds(16 * k, 16)] = dstlb[pl.ds(CHUNK + 16 * k, 16)]
            ae1b[pl.ds(16 * k, 16)] = ae1b[pl.ds(CHUNK + 16 * k, 16)]
            ae2b[pl.ds(16 * k, 16)] = ae2b[pl.ds(CHUNK + 16 * k, 16)]
            return 0

        lax.fori_loop(0, CHUNK // 16, shift_body, 0)
        _sentinel_fill(srcb, dstlb, ae1b, ae2b, CHUNK, CAPB)

    def flush(curv, nf):
        do = curv[0] >= CHUNK

        @pl.when(do)
        def _():
            do_flush(nf)

        curv = jnp.where(do, curv - CHUNK, curv)
        nf = jnp.where(do, nf + 1, nf)
        return curv, nf

    def pair_body(p, carry):
        curv, nf = carry
        for b in range(2):
            c = 2 * p + b
            drain(c, b)

            def sub_body(s, carry, b=b):
                cv, snf = carry

                def vec_body(k, cv, b=b):
                    i = s * SUB + k
                    srcv = srccs[b][pl.ds(16 * i, 16)]
                    dstv = dstcs[b][pl.ds(16 * i, 16)]
                    ae1v = ae1cs[b][pl.ds(16 * i, 16)]
                    ae2v = ae2cs[b][pl.ds(16 * i, 16)]
                    dstlv = dstv - nbase
                    m = (dstv >= nbase) & (dstlv < RNG)
                    pos = cv + jnp.cumsum(m.astype(jnp.int32)) - 1
                    pos = jnp.maximum(pos, 0)
                    plsc.store_scatter(srcb, [pos], srcv, mask=m)
                    plsc.store_scatter(dstlb, [pos], dstlv, mask=m)
                    plsc.store_scatter(ae1b, [pos], ae1v, mask=m)
                    plsc.store_scatter(ae2b, [pos], ae2v, mask=m)
                    return cv + plsc.all_reduce_population_count(m)

                cv = lax.fori_loop(0, SUB, vec_body, cv)
                return flush(cv, snf)

            curv, nf = lax.fori_loop(0, NSUB, sub_body, (curv, nf))

            @pl.when(c + 2 < NKIN)
            def _():
                fire(c + 2, b)
        return curv, nf

    curv = jnp.zeros((16,), jnp.int32)
    nf = jnp.int32(0)
    curv, nf = lax.fori_loop(0, NKIN // 2, pair_body, (curv, nf))

    for _ in range(2):
        do = curv[0] > 0

        @pl.when(do)
        def _():
            do_flush(nf)

        curv = jnp.where(do, jnp.maximum(curv - CHUNK, 0), curv)
        nf = jnp.where(do, nf + 1, nf)

    cvec[...] = jnp.zeros((16,), jnp.int32) + nf
    pltpu.sync_copy(cvec, nch.at[pl.ds(wid * 16, 16)])


_filter_call = pl.kernel(
    _filter_body,
    out_type=(
        jax.ShapeDtypeStruct((NW * ECAP,), jnp.int32),
        jax.ShapeDtypeStruct((NW * ECAP,), jnp.int32),
        jax.ShapeDtypeStruct((NW * ECAP,), jnp.float32),
        jax.ShapeDtypeStruct((NW * ECAP,), jnp.float32),
        jax.ShapeDtypeStruct((NW * 16,), jnp.int32),
    ),
    mesh=_mesh,
    compiler_params=pltpu.CompilerParams(needs_layout_passes=False),
    scratch_types=[
        pltpu.VMEM((KIN,), jnp.int32),
        pltpu.VMEM((KIN,), jnp.int32),
        pltpu.VMEM((KIN,), jnp.int32),
        pltpu.VMEM((KIN,), jnp.int32),
        pltpu.VMEM((KIN,), jnp.float32),
        pltpu.VMEM((KIN,), jnp.float32),
        pltpu.VMEM((KIN,), jnp.float32),
        pltpu.VMEM((KIN,), jnp.float32),
        pltpu.VMEM((CAPB,), jnp.int32),
        pltpu.VMEM((CAPB,), jnp.int32),
        pltpu.VMEM((CAPB,), jnp.float32),
        pltpu.VMEM((CAPB,), jnp.float32),
        pltpu.VMEM((16,), jnp.int32),
        pltpu.SemaphoreType.DMA,
    ],
)


def _layer_body(h, als, aldp, src_s, dstl_s, ae_s, nch, acc_o, den_o,
                als_t, ald_l, acc, den_l, srcfl, srcb, dstlb, aeb, exb,
                rows0, rows1, rows2, rows3, cvec, sem0, sem1, sem2, sem3):
    rows_bufs = (rows0, rows1, rows2, rows3)
    sems = (sem0, sem1, sem2, sem3)
    wid = _wid()
    nbase = wid * RNG
    abase = (nbase // 8) * 8
    adj = nbase - abase
    obase = wid * ECAP

    pltpu.sync_copy(als, als_t)
    pltpu.sync_copy(aldp.at[pl.ds(abase, RPAD + 8)], ald_l)
    pltpu.sync_copy(nch.at[pl.ds(wid * 16, 16)], cvec)
    nchunks = cvec[...][0]

    z16 = jnp.zeros((16,), jnp.float32)

    def zero_body(r, _):
        for v in range(8):
            acc[pl.ds(r * D + 16 * v, 16)] = z16
        den_l[pl.ds(r * 16, 16)] = z16
        return 0

    lax.fori_loop(0, RPAD, zero_body, 0)

    def chunk_body(c, _):
        o = obase + c * CHUNK
        pltpu.make_async_copy(src_s.at[pl.ds(o, CHUNK)], srcfl, sem0).start()
        pltpu.make_async_copy(dstl_s.at[pl.ds(o, CHUNK)], dstlb, sem1).start()
        pltpu.make_async_copy(ae_s.at[pl.ds(o, CHUNK)], aeb, sem2).start()
        pltpu.make_async_copy(src_s.at[pl.ds(o, CHUNK)], srcfl, sem0).wait()
        pltpu.make_async_copy(dstl_s.at[pl.ds(o, CHUNK)], dstlb, sem1).wait()
        pltpu.make_async_copy(ae_s.at[pl.ds(o, CHUNK)], aeb, sem2).wait()

        for k in range(CHUNK // 16):
            srcb[k // 8, pl.ds(16 * (k % 8), 16)] = srcfl[pl.ds(16 * k, 16)]

        for j in range(CROWS):
            pltpu.make_async_copy(
                h.at[srcb.at[j]], rows_bufs[j], sems[j]).start()

        def ex_body(i, _):
            srcv = srcfl[pl.ds(16 * i, 16)]
            dstlv = dstlb[pl.ds(16 * i, 16)]
            aev = aeb[pl.ds(16 * i, 16)]
            alsv = plsc.load_gather(als_t, [srcv])
            aldv = plsc.load_gather(ald_l, [dstlv + adj])
            a = alsv + aldv + aev
            a = jnp.where(a > 0, a, 0.2 * a)
            exb[pl.ds(16 * i, 16)] = jnp.exp(a)
            return 0

        lax.fori_loop(0, CHUNK // 16, ex_body, 0)

        for j in range(CROWS):
            pltpu.make_async_copy(
                h.at[srcb.at[j]], rows_bufs[j], sems[j]).wait()
            rj = rows_bufs[j]

            def edge_block(i, _, j=j, rj=rj):
                dv = dstlb[pl.ds(128 * j + 16 * i, 16)]
                sv = exb[pl.ds(128 * j + 16 * i, 16)]
                for lane in range(16):
                    e = 16 * i + lane
                    d = dv[lane]
                    s = sv[lane]
                    for v in range(8):
                        plsc.addupdate(acc.at[pl.ds(d * D + 16 * v, 16)],
                                       rj[e, pl.ds(16 * v, 16)] * s)
                    plsc.addupdate(den_l.at[pl.ds(d * 16, 16)],
                                   jnp.zeros((16,), jnp.float32) + s)
                return 0

            lax.fori_loop(0, 128 // 16, edge_block, 0)
        return 0

    lax.fori_loop(0, nchunks, chunk_body, 0)

    pltpu.sync_copy(acc, acc_o.at[pl.ds(wid * RPAD * D, RPAD * D)])
    pltpu.sync_copy(den_l, den_o.at[pl.ds(wid * RPAD * 16, RPAD * 16)])


_layer_call = pl.kernel(
    _layer_body,
    out_type=(
        jax.ShapeDtypeStruct((NW * RPAD * D,), jnp.float32),
        jax.ShapeDtypeStruct((NW * RPAD * 16,), jnp.float32),
    ),
    mesh=_mesh,
    compiler_params=pltpu.CompilerParams(needs_layout_passes=False),
    scratch_types=[
        pltpu.VMEM((N,), jnp.float32),
        pltpu.VMEM((RPAD + 8,), jnp.float32),
        pltpu.VMEM((RPAD * D,), jnp.float32),
        pltpu.VMEM((RPAD * 16,), jnp.float32),
        pltpu.VMEM((CHUNK,), jnp.int32),
        pltpu.VMEM((CROWS, 128), jnp.int32),
        pltpu.VMEM((CHUNK,), jnp.int32),
        pltpu.VMEM((CHUNK,), jnp.float32),
        pltpu.VMEM((CHUNK,), jnp.float32),
        pltpu.VMEM((128, D), jnp.float32),
        pltpu.VMEM((128, D), jnp.float32),
        pltpu.VMEM((128, D), jnp.float32),
        pltpu.VMEM((128, D), jnp.float32),
        pltpu.VMEM((16,), jnp.int32),
        pltpu.SemaphoreType.DMA,
        pltpu.SemaphoreType.DMA,
        pltpu.SemaphoreType.DMA,
        pltpu.SemaphoreType.DMA,
    ],
)


def _assemble(acc, den):
    accr = acc.reshape(NW, RPAD, D)[:, :RNG].reshape(NW * RNG, D)[:N]
    denr = den.reshape(NW, RPAD, 16)[:, :RNG, 0].reshape(NW * RNG)[:N]
    return accr, denr


def kernel(x, edge_index, edge_att, W1, a_src1, a_dst1, We1, a_e1, b1, W2,
           a_src2, a_dst2, We2, a_e2, b2, bn1_w, bn1_b, bn2_w, bn2_b):
    ale1, ale2 = _edge_al(edge_att, We1, a_e1, We2, a_e2)
    src_s, dstl_s, ae1_s, ae2_s, nch = _filter_call(
        edge_index[0], edge_index[1], ale1, ale2)

    h1, als1, ald1 = _pre(x, W1, a_src1, a_dst1)
    ald1p = jnp.pad(ald1[:, 0], (0, NPADDED - N))
    acc1, den1 = _layer_call(h1, als1[:, 0], ald1p, src_s, dstl_s, ae1_s, nch)
    acc1r, den1r = _assemble(acc1, den1)
    y1 = _post(acc1r, den1r, b1, bn1_w, bn1_b, relu=True)

    h2, als2, ald2 = _pre(y1, W2, a_src2, a_dst2)
    ald2p = jnp.pad(ald2[:, 0], (0, NPADDED - N))
    acc2, den2 = _layer_call(h2, als2[:, 0], ald2p, src_s, dstl_s, ae2_s, nch)
    acc2r, den2r = _assemble(acc2, den2)
    return _post(acc2r, den2r, b2, bn2_w, bn2_b, relu=False)
